# baseline (speedup 1.0000x reference)
import functools

import jax
import jax.numpy as jnp
from jax import lax
from jax.experimental import pallas as pl
from jax.experimental.pallas import tpu as pltpu
from jax.experimental.pallas import tpu_sc as plsc

N = 8192
G = 16
KNN = 5
ROWS = 256
NB = N // ROWS
BIG = 1e30
BIGI = 2**30

_NC, _NS = 2, 16
_NW = _NC * _NS
_CH = 128


W = 512
PAD = 128
KR = 128
KNB = N // KR


def _knn_body(bounds_ref, f_ref, fT_ref, br_ref, bc_ref, idx_ref,
              rv_ref, rc_ref):
    i = pl.program_id(0)
    r = f_ref[...]
    sq_r = jnp.sum(r * r, axis=1, keepdims=True)
    br = br_ref[...]
    rv_ref[...] = jnp.full((KR, PAD), BIG, jnp.float32)
    rc_ref[...] = jnp.full((KR, PAD), BIGI, jnp.int32)
    c0 = bounds_ref[i, 0]
    c1 = bounds_ref[i, 1]

    def chunk(c, _):
        cT = fT_ref[:, pl.ds(c * W, W)]
        d = jnp.dot(r, cT, preferred_element_type=jnp.float32) * -2.0
        d = d + sq_r
        d = d + jnp.sum(cT * cT, axis=0, keepdims=True)
        d = jnp.where(br == bc_ref[:, pl.ds(c * W, W)], d, BIG)
        col = c * W + lax.broadcasted_iota(jnp.int32, (KR, W), 1)
        vals = jnp.concatenate([rv_ref[...], d], axis=1)
        cols = jnp.concatenate([rc_ref[...], col], axis=1)
        ms, its = [], []
        for _ in range(KNN):
            m = jnp.min(vals, axis=1, keepdims=True)
            it = jnp.min(jnp.where(vals == m, cols, BIGI), axis=1,
                         keepdims=True)
            ms.append(m)
            its.append(it)
            vals = jnp.where(cols == it, float("inf"), vals)
        padv = jnp.full((KR, PAD - KNN), BIG, jnp.float32)
        padc = jnp.full((KR, PAD - KNN), BIGI, jnp.int32)
        rv_ref[...] = jnp.concatenate(ms + [padv], axis=1)
        rc_ref[...] = jnp.concatenate(its + [padc], axis=1)
        return 0

    lax.fori_loop(c0, c1, chunk, 0)
    idx_ref[...] = rc_ref[:, :KNN]


def _knn_call(f, fT, br, bc, bounds, feat):
    return pl.pallas_call(
        _knn_body,
        grid=(KNB,),
        in_specs=[
            pl.BlockSpec(memory_space=pltpu.SMEM),
            pl.BlockSpec((KR, feat), lambda i: (i, 0)),
            pl.BlockSpec((feat, N), lambda i: (0, 0)),
            pl.BlockSpec((KR, 1), lambda i: (i, 0)),
            pl.BlockSpec((1, N), lambda i: (0, 0)),
        ],
        out_specs=pl.BlockSpec((KR, KNN), lambda i: (i, 0)),
        out_shape=jax.ShapeDtypeStruct((N, KNN), jnp.int32),
        scratch_shapes=[
            pltpu.VMEM((KR, PAD), jnp.float32),
            pltpu.VMEM((KR, PAD), jnp.int32),
        ],
    )(bounds, f, fT, br, bc)


def _sc_gather(table, idx_flat, depth, total):
    per_w = total // _NW
    nchunk = per_w // _CH
    mesh = plsc.VectorSubcoreMesh(core_axis_name="c", subcore_axis_name="s",
                                  num_cores=_NC, num_subcores=_NS)

    @functools.partial(
        pl.kernel,
        out_type=jax.ShapeDtypeStruct((total, depth), jnp.float32),
        mesh=mesh,
        compiler_params=pltpu.CompilerParams(use_tc_tiling_on_sc=False),
        scratch_types=[
            pltpu.VMEM((_CH,), jnp.int32),
            pltpu.VMEM((_CH, depth), jnp.float32),
            pltpu.SemaphoreType.DMA,
        ],
    )
    def gather_k(table_hbm, idx_hbm, out_hbm, idx_v, rows_v, sem):
        wid = lax.axis_index("s") * _NC + lax.axis_index("c")
        base = wid * per_w
        for j in range(nchunk):
            off = base + j * _CH
            pltpu.sync_copy(idx_hbm.at[pl.ds(off, _CH)], idx_v)
            pltpu.async_copy(table_hbm.at[idx_v], rows_v, sem).wait()
            pltpu.sync_copy(rows_v, out_hbm.at[pl.ds(off, _CH)])

    return gather_k(table, idx_flat)


def _conv1_body(x_ref, g_ref, w1_ref, b1_ref, w2_ref, b2_ref, w3_ref, b3_ref,
                x1_ref):
    xi = x_ref[...]
    acc = jnp.zeros((ROWS, 64), jnp.float32)
    for k in range(KNN):
        xj = g_ref[k][:, :4]
        e = jnp.concatenate([xi, xj - xi], axis=1)
        h = jnp.maximum(
            jnp.dot(e, w1_ref[...], preferred_element_type=jnp.float32)
            + b1_ref[...], 0.0)
        h = jnp.maximum(
            jnp.dot(h, w2_ref[...], preferred_element_type=jnp.float32)
            + b2_ref[...], 0.0)
        h = jnp.maximum(
            jnp.dot(h, w3_ref[...], preferred_element_type=jnp.float32)
            + b3_ref[...], 0.0)
        acc = acc + h
    x1_ref[...] = acc


def _conv1_call(xx, g1, W1, b1, W2, b2, W3, b3):
    return pl.pallas_call(
        _conv1_body,
        grid=(NB,),
        in_specs=[
            pl.BlockSpec((ROWS, 4), lambda i: (i, 0)),
            pl.BlockSpec((KNN, ROWS, 8), lambda i: (0, i, 0)),
            pl.BlockSpec((8, 64), lambda i: (0, 0)),
            pl.BlockSpec((1, 64), lambda i: (0, 0)),
            pl.BlockSpec((64, 64), lambda i: (0, 0)),
            pl.BlockSpec((1, 64), lambda i: (0, 0)),
            pl.BlockSpec((64, 64), lambda i: (0, 0)),
            pl.BlockSpec((1, 64), lambda i: (0, 0)),
        ],
        out_specs=pl.BlockSpec((ROWS, 64), lambda i: (i, 0)),
        out_shape=jax.ShapeDtypeStruct((N, 64), jnp.float32),
    )(xx, g1, W1, b1, W2, b2, W3, b3)


def _conv2_body(gb_ref, x1_ref, g2_ref, w4_ref, b4_ref, wl_ref, bl_ref,
                br_ref, wm1_ref, bm1_ref, wm2_ref, bm2_ref, wm3_ref, bm3_ref,
                o_ref, pooled_ref):
    i = pl.program_id(0)

    @pl.when(i == 0)
    def _():
        pooled_ref[...] = jnp.full((G, 1024), -jnp.inf, jnp.float32)

    xi = x1_ref[...]
    x2 = jnp.zeros((ROWS, 128), jnp.float32)
    for k in range(KNN):
        e = jnp.concatenate([xi, g2_ref[k] - xi], axis=1)
        x2 = x2 + jnp.maximum(
            jnp.dot(e, w4_ref[...], preferred_element_type=jnp.float32)
            + b4_ref[...], 0.0)
    cat = jnp.concatenate([xi, x2], axis=1)
    out = jnp.dot(cat, wl_ref[...], preferred_element_type=jnp.float32)
    out = out + bl_ref[...]
    b = br_ref[...]
    glo = gb_ref[i, 0]
    ghi = gb_ref[i, 1]
    neg = float("-inf")
    for g in range(G):
        @pl.when((glo <= g) & (g <= ghi))
        def _(g=g):
            m = jnp.max(jnp.where(b == g, out, neg), axis=0, keepdims=True)
            pooled_ref[pl.ds(g, 1), :] = jnp.maximum(
                pooled_ref[pl.ds(g, 1), :], m)

    @pl.when(i == NB - 1)
    def _():
        h = jnp.maximum(
            jnp.dot(pooled_ref[...], wm1_ref[...],
                    preferred_element_type=jnp.float32) + bm1_ref[...], 0.0)
        h = jnp.maximum(
            jnp.dot(h, wm2_ref[...], preferred_element_type=jnp.float32)
            + bm2_ref[...], 0.0)
        o_ref[...] = (
            jnp.dot(h, wm3_ref[...], preferred_element_type=jnp.float32)
            + bm3_ref[...])


def _conv2_call(gb, x1, g2, W4, b4, Wl, bl, br, Wm1, bm1, Wm2, bm2, Wm3, bm3):
    return pl.pallas_call(
        _conv2_body,
        grid=(NB,),
        in_specs=[
            pl.BlockSpec(memory_space=pltpu.SMEM),
            pl.BlockSpec((ROWS, 64), lambda i: (i, 0)),
            pl.BlockSpec((KNN, ROWS, 64), lambda i: (0, i, 0)),
            pl.BlockSpec((128, 128), lambda i: (0, 0)),
            pl.BlockSpec((1, 128), lambda i: (0, 0)),
            pl.BlockSpec((192, 1024), lambda i: (0, 0)),
            pl.BlockSpec((1, 1024), lambda i: (0, 0)),
            pl.BlockSpec((ROWS, 1), lambda i: (i, 0)),
            pl.BlockSpec((1024, 512), lambda i: (0, 0)),
            pl.BlockSpec((1, 512), lambda i: (0, 0)),
            pl.BlockSpec((512, 256), lambda i: (0, 0)),
            pl.BlockSpec((1, 256), lambda i: (0, 0)),
            pl.BlockSpec((256, 40), lambda i: (0, 0)),
            pl.BlockSpec((1, 40), lambda i: (0, 0)),
        ],
        out_specs=pl.BlockSpec((G, 40), lambda i: (0, 0)),
        out_shape=jax.ShapeDtypeStruct((G, 40), jnp.float32),
        scratch_shapes=[pltpu.VMEM((G, 1024), jnp.float32)],
    )(gb, x1, g2, W4, b4, Wl, bl, br, Wm1, bm1, Wm2, bm2, Wm3, bm3)


def kernel(x, pos, batch, W1, b1, W2, b2, W3, b3, W4, b4, Wl, bl,
           Wm1, bm1, Wm2, bm2, Wm3, bm3):
    xx = jnp.concatenate([x, pos], axis=1)
    batch = batch.astype(jnp.int32)
    br = batch.reshape(N, 1)
    bc = batch.reshape(1, N)

    seg_lo = jnp.searchsorted(batch, jnp.arange(G, dtype=jnp.int32))
    seg_hi = jnp.searchsorted(batch, jnp.arange(G, dtype=jnp.int32),
                              side="right")
    lo = seg_lo[batch[::KR]].astype(jnp.int32)
    hi = seg_hi[batch[KR - 1::KR]].astype(jnp.int32)
    bounds = jnp.stack([lo // W, (hi + W - 1) // W], axis=1).astype(jnp.int32)

    idx1 = _knn_call(xx, xx.T, br, bc, bounds, 4)
    xx8 = jnp.pad(xx, ((0, 0), (0, 4)))
    g1 = _sc_gather(xx8, idx1.T.reshape(-1), 8, KNN * N).reshape(KNN, N, 8)
    x1 = _conv1_call(xx, g1, W1, b1.reshape(1, 64), W2, b2.reshape(1, 64),
                     W3, b3.reshape(1, 64))

    idx2 = _knn_call(x1, x1.T, br, bc, bounds, 64)
    g2 = _sc_gather(x1, idx2.T.reshape(-1), 64, KNN * N).reshape(KNN, N, 64)

    gb = jnp.stack([batch[::ROWS], batch[ROWS - 1::ROWS]],
                   axis=1).astype(jnp.int32)
    return _conv2_call(gb, x1, g2, W4, b4.reshape(1, 128), Wl,
                       bl.reshape(1, 1024), br, Wm1, bm1.reshape(1, 512),
                       Wm2, bm2.reshape(1, 256), Wm3, bm3.reshape(1, 40))


# speedup vs baseline: 1.3677x; 1.3677x over previous
import functools

import jax
import jax.numpy as jnp
from jax import lax
from jax.experimental import pallas as pl
from jax.experimental.pallas import tpu as pltpu
from jax.experimental.pallas import tpu_sc as plsc

N = 8192
G = 16
KNN = 5
ROWS = 256
NB = N // ROWS
BIG = 1e30
BIGI = 2**30

_NC, _NS = 2, 16
_NW = _NC * _NS
_CH = 128


W = 512
PAD = 128
KR = 256
KNB = N // KR


def _knn_body(bounds_ref, f_ref, fT_ref, br_ref, bc_ref, idx_ref,
              rv_ref, rc_ref):
    i = pl.program_id(0)
    r = f_ref[...]
    sq_r = jnp.sum(r * r, axis=1, keepdims=True)
    br = br_ref[...]
    rv_ref[...] = jnp.full((KR, PAD), BIG, jnp.float32)
    rc_ref[...] = jnp.full((KR, PAD), BIGI, jnp.int32)
    c0 = bounds_ref[i, 0]
    c1 = bounds_ref[i, 1]

    def chunk(c, _):
        cT = fT_ref[:, pl.ds(c * W, W)]
        d = jnp.dot(r, cT, preferred_element_type=jnp.float32) * -2.0
        d = d + sq_r
        d = d + jnp.sum(cT * cT, axis=0, keepdims=True)
        d = jnp.where(br == bc_ref[:, pl.ds(c * W, W)], d, BIG)
        col = c * W + lax.broadcasted_iota(jnp.int32, (KR, W), 1)
        vals = jnp.concatenate([rv_ref[...], d], axis=1)
        cols = jnp.concatenate([rc_ref[...], col], axis=1)
        ms, its = [], []
        for _ in range(KNN):
            m = jnp.min(vals, axis=1, keepdims=True)
            it = jnp.min(jnp.where(vals == m, cols, BIGI), axis=1,
                         keepdims=True)
            ms.append(m)
            its.append(it)
            vals = jnp.where(cols == it, float("inf"), vals)
        padv = jnp.full((KR, PAD - KNN), BIG, jnp.float32)
        padc = jnp.full((KR, PAD - KNN), BIGI, jnp.int32)
        rv_ref[...] = jnp.concatenate(ms + [padv], axis=1)
        rc_ref[...] = jnp.concatenate(its + [padc], axis=1)
        return 0

    lax.fori_loop(c0, c1, chunk, 0)
    idx_ref[...] = rc_ref[:, :KNN]


def _knn_call(f, fT, br, bc, bounds, feat):
    return pl.pallas_call(
        _knn_body,
        grid=(KNB,),
        in_specs=[
            pl.BlockSpec(memory_space=pltpu.SMEM),
            pl.BlockSpec((KR, feat), lambda i: (i, 0)),
            pl.BlockSpec((feat, N), lambda i: (0, 0)),
            pl.BlockSpec((KR, 1), lambda i: (i, 0)),
            pl.BlockSpec((1, N), lambda i: (0, 0)),
        ],
        out_specs=pl.BlockSpec((KR, KNN), lambda i: (i, 0)),
        out_shape=jax.ShapeDtypeStruct((N, KNN), jnp.int32),
        scratch_shapes=[
            pltpu.VMEM((KR, PAD), jnp.float32),
            pltpu.VMEM((KR, PAD), jnp.int32),
        ],
    )(bounds, f, fT, br, bc)


def _sc_gather(table, idx_flat, depth, total):
    per_w = total // _NW
    nchunk = per_w // _CH
    mesh = plsc.VectorSubcoreMesh(core_axis_name="c", subcore_axis_name="s",
                                  num_cores=_NC, num_subcores=_NS)

    @functools.partial(
        pl.kernel,
        out_type=jax.ShapeDtypeStruct((total, depth), jnp.float32),
        mesh=mesh,
        compiler_params=pltpu.CompilerParams(use_tc_tiling_on_sc=False),
        scratch_types=[
            pltpu.VMEM((_CH,), jnp.int32),
            pltpu.VMEM((_CH, depth), jnp.float32),
            pltpu.SemaphoreType.DMA,
        ],
    )
    def gather_k(table_hbm, idx_hbm, out_hbm, idx_v, rows_v, sem):
        wid = lax.axis_index("s") * _NC + lax.axis_index("c")
        base = wid * per_w
        for j in range(nchunk):
            off = base + j * _CH
            pltpu.sync_copy(idx_hbm.at[pl.ds(off, _CH)], idx_v)
            pltpu.async_copy(table_hbm.at[idx_v], rows_v, sem).wait()
            pltpu.sync_copy(rows_v, out_hbm.at[pl.ds(off, _CH)])

    return gather_k(table, idx_flat)


def _conv1_body(x_ref, g_ref, w1_ref, b1_ref, w2_ref, b2_ref, w3_ref, b3_ref,
                x1_ref):
    xi = x_ref[...]
    acc = jnp.zeros((ROWS, 64), jnp.float32)
    for k in range(KNN):
        xj = g_ref[k][:, :4]
        e = jnp.concatenate([xi, xj - xi], axis=1)
        h = jnp.maximum(
            jnp.dot(e, w1_ref[...], preferred_element_type=jnp.float32)
            + b1_ref[...], 0.0)
        h = jnp.maximum(
            jnp.dot(h, w2_ref[...], preferred_element_type=jnp.float32)
            + b2_ref[...], 0.0)
        h = jnp.maximum(
            jnp.dot(h, w3_ref[...], preferred_element_type=jnp.float32)
            + b3_ref[...], 0.0)
        acc = acc + h
    x1_ref[...] = acc


def _conv1_call(xx, g1, W1, b1, W2, b2, W3, b3):
    return pl.pallas_call(
        _conv1_body,
        grid=(NB,),
        in_specs=[
            pl.BlockSpec((ROWS, 4), lambda i: (i, 0)),
            pl.BlockSpec((KNN, ROWS, 8), lambda i: (0, i, 0)),
            pl.BlockSpec((8, 64), lambda i: (0, 0)),
            pl.BlockSpec((1, 64), lambda i: (0, 0)),
            pl.BlockSpec((64, 64), lambda i: (0, 0)),
            pl.BlockSpec((1, 64), lambda i: (0, 0)),
            pl.BlockSpec((64, 64), lambda i: (0, 0)),
            pl.BlockSpec((1, 64), lambda i: (0, 0)),
        ],
        out_specs=pl.BlockSpec((ROWS, 64), lambda i: (i, 0)),
        out_shape=jax.ShapeDtypeStruct((N, 64), jnp.float32),
    )(xx, g1, W1, b1, W2, b2, W3, b3)


def _conv2_body(gb_ref, x1_ref, g2_ref, w4_ref, b4_ref, wl_ref, bl_ref,
                br_ref, wm1_ref, bm1_ref, wm2_ref, bm2_ref, wm3_ref, bm3_ref,
                o_ref, pooled_ref):
    i = pl.program_id(0)

    @pl.when(i == 0)
    def _():
        pooled_ref[...] = jnp.full((G, 1024), -jnp.inf, jnp.float32)

    xi = x1_ref[...]
    x2 = jnp.zeros((ROWS, 128), jnp.float32)
    for k in range(KNN):
        e = jnp.concatenate([xi, g2_ref[k] - xi], axis=1)
        x2 = x2 + jnp.maximum(
            jnp.dot(e, w4_ref[...], preferred_element_type=jnp.float32)
            + b4_ref[...], 0.0)
    cat = jnp.concatenate([xi, x2], axis=1)
    out = jnp.dot(cat, wl_ref[...], preferred_element_type=jnp.float32)
    out = out + bl_ref[...]
    b = br_ref[...]
    glo = gb_ref[i, 0]
    ghi = gb_ref[i, 1]
    neg = float("-inf")
    for g in range(G):
        @pl.when((glo <= g) & (g <= ghi))
        def _(g=g):
            m = jnp.max(jnp.where(b == g, out, neg), axis=0, keepdims=True)
            pooled_ref[pl.ds(g, 1), :] = jnp.maximum(
                pooled_ref[pl.ds(g, 1), :], m)

    @pl.when(i == NB - 1)
    def _():
        h = jnp.maximum(
            jnp.dot(pooled_ref[...], wm1_ref[...],
                    preferred_element_type=jnp.float32) + bm1_ref[...], 0.0)
        h = jnp.maximum(
            jnp.dot(h, wm2_ref[...], preferred_element_type=jnp.float32)
            + bm2_ref[...], 0.0)
        o_ref[...] = (
            jnp.dot(h, wm3_ref[...], preferred_element_type=jnp.float32)
            + bm3_ref[...])


def _conv2_call(gb, x1, g2, W4, b4, Wl, bl, br, Wm1, bm1, Wm2, bm2, Wm3, bm3):
    return pl.pallas_call(
        _conv2_body,
        grid=(NB,),
        in_specs=[
            pl.BlockSpec(memory_space=pltpu.SMEM),
            pl.BlockSpec((ROWS, 64), lambda i: (i, 0)),
            pl.BlockSpec((KNN, ROWS, 64), lambda i: (0, i, 0)),
            pl.BlockSpec((128, 128), lambda i: (0, 0)),
            pl.BlockSpec((1, 128), lambda i: (0, 0)),
            pl.BlockSpec((192, 1024), lambda i: (0, 0)),
            pl.BlockSpec((1, 1024), lambda i: (0, 0)),
            pl.BlockSpec((ROWS, 1), lambda i: (i, 0)),
            pl.BlockSpec((1024, 512), lambda i: (0, 0)),
            pl.BlockSpec((1, 512), lambda i: (0, 0)),
            pl.BlockSpec((512, 256), lambda i: (0, 0)),
            pl.BlockSpec((1, 256), lambda i: (0, 0)),
            pl.BlockSpec((256, 40), lambda i: (0, 0)),
            pl.BlockSpec((1, 40), lambda i: (0, 0)),
        ],
        out_specs=pl.BlockSpec((G, 40), lambda i: (0, 0)),
        out_shape=jax.ShapeDtypeStruct((G, 40), jnp.float32),
        scratch_shapes=[pltpu.VMEM((G, 1024), jnp.float32)],
    )(gb, x1, g2, W4, b4, Wl, bl, br, Wm1, bm1, Wm2, bm2, Wm3, bm3)


def kernel(x, pos, batch, W1, b1, W2, b2, W3, b3, W4, b4, Wl, bl,
           Wm1, bm1, Wm2, bm2, Wm3, bm3):
    xx = jnp.concatenate([x, pos], axis=1)
    batch = batch.astype(jnp.int32)
    br = batch.reshape(N, 1)
    bc = batch.reshape(1, N)

    seg_lo = jnp.searchsorted(batch, jnp.arange(G, dtype=jnp.int32))
    seg_hi = jnp.searchsorted(batch, jnp.arange(G, dtype=jnp.int32),
                              side="right")
    lo = seg_lo[batch[::KR]].astype(jnp.int32)
    hi = seg_hi[batch[KR - 1::KR]].astype(jnp.int32)
    bounds = jnp.stack([lo // W, (hi + W - 1) // W], axis=1).astype(jnp.int32)

    idx1 = _knn_call(xx, xx.T, br, bc, bounds, 4)
    xx8 = jnp.pad(xx, ((0, 0), (0, 4)))
    g1 = _sc_gather(xx8, idx1.T.reshape(-1), 8, KNN * N).reshape(KNN, N, 8)
    x1 = _conv1_call(xx, g1, W1, b1.reshape(1, 64), W2, b2.reshape(1, 64),
                     W3, b3.reshape(1, 64))

    idx2 = _knn_call(x1, x1.T, br, bc, bounds, 64)
    g2 = _sc_gather(x1, idx2.T.reshape(-1), 64, KNN * N).reshape(KNN, N, 64)

    gb = jnp.stack([batch[::ROWS], batch[ROWS - 1::ROWS]],
                   axis=1).astype(jnp.int32)
    return _conv2_call(gb, x1, g2, W4, b4.reshape(1, 128), Wl,
                       bl.reshape(1, 1024), br, Wm1, bm1.reshape(1, 512),
                       Wm2, bm2.reshape(1, 256), Wm3, bm3.reshape(1, 40))


# speedup vs baseline: 1.3682x; 1.0004x over previous
import functools

import jax
import jax.numpy as jnp
from jax import lax
from jax.experimental import pallas as pl
from jax.experimental.pallas import tpu as pltpu
from jax.experimental.pallas import tpu_sc as plsc

N = 8192
G = 16
KNN = 5
ROWS = 256
NB = N // ROWS
BIG = 1e30
BIGI = 2**30

_NC, _NS = 2, 16
_NW = _NC * _NS
_CH = 128


W = 512
PAD = 128
KR = 256
KNB = N // KR


def _knn_body(bounds_ref, f_ref, fT_ref, br_ref, bc_ref, idx_ref,
              rv_ref, rc_ref):
    i = pl.program_id(0)
    r = f_ref[...]
    sq_r = jnp.sum(r * r, axis=1, keepdims=True)
    br = br_ref[...]
    rv_ref[...] = jnp.full((KR, PAD), BIG, jnp.float32)
    rc_ref[...] = jnp.full((KR, PAD), BIGI, jnp.int32)
    c0 = bounds_ref[i, 0]
    c1 = bounds_ref[i, 1]

    def chunk(c, _):
        cT = fT_ref[:, pl.ds(c * W, W)]
        d = jnp.dot(r, cT, preferred_element_type=jnp.float32) * -2.0
        d = d + sq_r
        d = d + jnp.sum(cT * cT, axis=0, keepdims=True)
        d = jnp.where(br == bc_ref[:, pl.ds(c * W, W)], d, BIG)
        col = c * W + lax.broadcasted_iota(jnp.int32, (KR, W), 1)
        vals = jnp.concatenate([rv_ref[...], d], axis=1)
        cols = jnp.concatenate([rc_ref[...], col], axis=1)
        ms, its = [], []
        for _ in range(KNN):
            m = jnp.min(vals, axis=1, keepdims=True)
            it = jnp.min(jnp.where(vals == m, cols, BIGI), axis=1,
                         keepdims=True)
            ms.append(m)
            its.append(it)
            vals = jnp.where(cols == it, float("inf"), vals)
        padv = jnp.full((KR, PAD - KNN), BIG, jnp.float32)
        padc = jnp.full((KR, PAD - KNN), BIGI, jnp.int32)
        rv_ref[...] = jnp.concatenate(ms + [padv], axis=1)
        rc_ref[...] = jnp.concatenate(its + [padc], axis=1)
        return 0

    lax.fori_loop(c0, c1, chunk, 0)
    idx_ref[...] = rc_ref[:, :KNN]


def _knn_call(f, fT, br, bc, bounds, feat):
    return pl.pallas_call(
        _knn_body,
        grid=(KNB,),
        in_specs=[
            pl.BlockSpec(memory_space=pltpu.SMEM),
            pl.BlockSpec((KR, feat), lambda i: (i, 0)),
            pl.BlockSpec((feat, N), lambda i: (0, 0)),
            pl.BlockSpec((KR, 1), lambda i: (i, 0)),
            pl.BlockSpec((1, N), lambda i: (0, 0)),
        ],
        out_specs=pl.BlockSpec((KR, KNN), lambda i: (i, 0)),
        out_shape=jax.ShapeDtypeStruct((N, KNN), jnp.int32),
        scratch_shapes=[
            pltpu.VMEM((KR, PAD), jnp.float32),
            pltpu.VMEM((KR, PAD), jnp.int32),
        ],
    )(bounds, f, fT, br, bc)


def _sc_gather(table, idx_flat, depth, total):
    per_w = total // _NW
    nchunk = per_w // _CH
    mesh = plsc.VectorSubcoreMesh(core_axis_name="c", subcore_axis_name="s",
                                  num_cores=_NC, num_subcores=_NS)

    @functools.partial(
        pl.kernel,
        out_type=jax.ShapeDtypeStruct((total, depth), jnp.float32),
        mesh=mesh,
        compiler_params=pltpu.CompilerParams(use_tc_tiling_on_sc=False),
        scratch_types=[
            pltpu.VMEM((_CH,), jnp.int32),
            pltpu.VMEM((_CH, depth), jnp.float32),
            pltpu.SemaphoreType.DMA,
        ],
    )
    def gather_k(table_hbm, idx_hbm, out_hbm, idx_v, rows_v, sem):
        wid = lax.axis_index("s") * _NC + lax.axis_index("c")
        base = wid * per_w
        for j in range(nchunk):
            off = base + j * _CH
            pltpu.sync_copy(idx_hbm.at[pl.ds(off, _CH)], idx_v)
            pltpu.async_copy(table_hbm.at[idx_v], rows_v, sem).wait()
            pltpu.sync_copy(rows_v, out_hbm.at[pl.ds(off, _CH)])

    return gather_k(table, idx_flat)


def _conv1_body(x_ref, g_ref, w1_ref, b1_ref, w2_ref, b2_ref, w3_ref, b3_ref,
                x1_ref):
    xi = x_ref[...]
    xi5 = jnp.concatenate([xi] * KNN, axis=0)
    xj = g_ref[...].reshape(KNN * ROWS, 8)[:, :4]
    e = jnp.concatenate([xi5, xj - xi5], axis=1)
    h = jnp.maximum(
        jnp.dot(e, w1_ref[...], preferred_element_type=jnp.float32)
        + b1_ref[...], 0.0)
    h = jnp.maximum(
        jnp.dot(h, w2_ref[...], preferred_element_type=jnp.float32)
        + b2_ref[...], 0.0)
    h = jnp.maximum(
        jnp.dot(h, w3_ref[...], preferred_element_type=jnp.float32)
        + b3_ref[...], 0.0)
    acc = h[0:ROWS]
    for k in range(1, KNN):
        acc = acc + h[k * ROWS:(k + 1) * ROWS]
    x1_ref[...] = acc


def _conv1_call(xx, g1, W1, b1, W2, b2, W3, b3):
    return pl.pallas_call(
        _conv1_body,
        grid=(NB,),
        in_specs=[
            pl.BlockSpec((ROWS, 4), lambda i: (i, 0)),
            pl.BlockSpec((KNN, ROWS, 8), lambda i: (0, i, 0)),
            pl.BlockSpec((8, 64), lambda i: (0, 0)),
            pl.BlockSpec((1, 64), lambda i: (0, 0)),
            pl.BlockSpec((64, 64), lambda i: (0, 0)),
            pl.BlockSpec((1, 64), lambda i: (0, 0)),
            pl.BlockSpec((64, 64), lambda i: (0, 0)),
            pl.BlockSpec((1, 64), lambda i: (0, 0)),
        ],
        out_specs=pl.BlockSpec((ROWS, 64), lambda i: (i, 0)),
        out_shape=jax.ShapeDtypeStruct((N, 64), jnp.float32),
    )(xx, g1, W1, b1, W2, b2, W3, b3)


def _conv2_body(gb_ref, x1_ref, g2_ref, w4_ref, b4_ref, wl_ref, bl_ref,
                br_ref, wm1_ref, bm1_ref, wm2_ref, bm2_ref, wm3_ref, bm3_ref,
                o_ref, pooled_ref):
    i = pl.program_id(0)

    @pl.when(i == 0)
    def _():
        pooled_ref[...] = jnp.full((G, 1024), -jnp.inf, jnp.float32)

    xi = x1_ref[...]
    xi5 = jnp.concatenate([xi] * KNN, axis=0)
    gall = g2_ref[...].reshape(KNN * ROWS, 64)
    e = jnp.concatenate([xi5, gall - xi5], axis=1)
    hh = jnp.maximum(
        jnp.dot(e, w4_ref[...], preferred_element_type=jnp.float32)
        + b4_ref[...], 0.0)
    x2 = hh[0:ROWS]
    for k in range(1, KNN):
        x2 = x2 + hh[k * ROWS:(k + 1) * ROWS]
    cat = jnp.concatenate([xi, x2], axis=1)
    out = jnp.dot(cat, wl_ref[...], preferred_element_type=jnp.float32)
    out = out + bl_ref[...]
    b = br_ref[...]
    glo = gb_ref[i, 0]
    ghi = gb_ref[i, 1]
    neg = float("-inf")
    for g in range(G):
        @pl.when((glo <= g) & (g <= ghi))
        def _(g=g):
            m = jnp.max(jnp.where(b == g, out, neg), axis=0, keepdims=True)
            pooled_ref[pl.ds(g, 1), :] = jnp.maximum(
                pooled_ref[pl.ds(g, 1), :], m)

    @pl.when(i == NB - 1)
    def _():
        h = jnp.maximum(
            jnp.dot(pooled_ref[...], wm1_ref[...],
                    preferred_element_type=jnp.float32) + bm1_ref[...], 0.0)
        h = jnp.maximum(
            jnp.dot(h, wm2_ref[...], preferred_element_type=jnp.float32)
            + bm2_ref[...], 0.0)
        o_ref[...] = (
            jnp.dot(h, wm3_ref[...], preferred_element_type=jnp.float32)
            + bm3_ref[...])


def _conv2_call(gb, x1, g2, W4, b4, Wl, bl, br, Wm1, bm1, Wm2, bm2, Wm3, bm3):
    return pl.pallas_call(
        _conv2_body,
        grid=(NB,),
        in_specs=[
            pl.BlockSpec(memory_space=pltpu.SMEM),
            pl.BlockSpec((ROWS, 64), lambda i: (i, 0)),
            pl.BlockSpec((KNN, ROWS, 64), lambda i: (0, i, 0)),
            pl.BlockSpec((128, 128), lambda i: (0, 0)),
            pl.BlockSpec((1, 128), lambda i: (0, 0)),
            pl.BlockSpec((192, 1024), lambda i: (0, 0)),
            pl.BlockSpec((1, 1024), lambda i: (0, 0)),
            pl.BlockSpec((ROWS, 1), lambda i: (i, 0)),
            pl.BlockSpec((1024, 512), lambda i: (0, 0)),
            pl.BlockSpec((1, 512), lambda i: (0, 0)),
            pl.BlockSpec((512, 256), lambda i: (0, 0)),
            pl.BlockSpec((1, 256), lambda i: (0, 0)),
            pl.BlockSpec((256, 40), lambda i: (0, 0)),
            pl.BlockSpec((1, 40), lambda i: (0, 0)),
        ],
        out_specs=pl.BlockSpec((G, 40), lambda i: (0, 0)),
        out_shape=jax.ShapeDtypeStruct((G, 40), jnp.float32),
        scratch_shapes=[pltpu.VMEM((G, 1024), jnp.float32)],
    )(gb, x1, g2, W4, b4, Wl, bl, br, Wm1, bm1, Wm2, bm2, Wm3, bm3)


def kernel(x, pos, batch, W1, b1, W2, b2, W3, b3, W4, b4, Wl, bl,
           Wm1, bm1, Wm2, bm2, Wm3, bm3):
    xx = jnp.concatenate([x, pos], axis=1)
    batch = batch.astype(jnp.int32)
    br = batch.reshape(N, 1)
    bc = batch.reshape(1, N)

    seg_lo = jnp.searchsorted(batch, jnp.arange(G, dtype=jnp.int32))
    seg_hi = jnp.searchsorted(batch, jnp.arange(G, dtype=jnp.int32),
                              side="right")
    lo = seg_lo[batch[::KR]].astype(jnp.int32)
    hi = seg_hi[batch[KR - 1::KR]].astype(jnp.int32)
    bounds = jnp.stack([lo // W, (hi + W - 1) // W], axis=1).astype(jnp.int32)

    idx1 = _knn_call(xx, xx.T, br, bc, bounds, 4)
    xx8 = jnp.pad(xx, ((0, 0), (0, 4)))
    g1 = _sc_gather(xx8, idx1.T.reshape(-1), 8, KNN * N).reshape(KNN, N, 8)
    x1 = _conv1_call(xx, g1, W1, b1.reshape(1, 64), W2, b2.reshape(1, 64),
                     W3, b3.reshape(1, 64))

    idx2 = _knn_call(x1, x1.T, br, bc, bounds, 64)
    g2 = _sc_gather(x1, idx2.T.reshape(-1), 64, KNN * N).reshape(KNN, N, 64)

    gb = jnp.stack([batch[::ROWS], batch[ROWS - 1::ROWS]],
                   axis=1).astype(jnp.int32)
    return _conv2_call(gb, x1, g2, W4, b4.reshape(1, 128), Wl,
                       bl.reshape(1, 1024), br, Wm1, bm1.reshape(1, 512),
                       Wm2, bm2.reshape(1, 256), Wm3, bm3.reshape(1, 40))
